# Initial kernel scaffold; baseline (speedup 1.0000x reference)
import jax
import jax.numpy as jnp
from jax.experimental import pallas as pl


def kernel(x, W_enc, b_enc, W_dec, b_dec):
    raise NotImplementedError("write your pallas kernel here")


# speedup vs baseline: 5.2819x; 5.2819x over previous
import jax
import jax.numpy as jnp
from jax import lax
from jax.experimental import pallas as pl
from jax.experimental.pallas import tpu as pltpu

_TOPK = 64
_ENC_ROWS = 16
_DEC_ROWS = 64


def _encode_topk_kernel(x_ref, we_ref, be_ref, zs_ref):
    x = x_ref[...]
    z = jnp.dot(x, we_ref[...], preferred_element_type=jnp.float32) + be_ref[...]
    b, d_lat = z.shape
    ai = lax.bitcast_convert_type(jnp.abs(z), jnp.int32)

    p = jnp.zeros((b, 1), jnp.int32)
    for bit in range(30, -1, -1):
        cand = p | jnp.int32(1 << bit)
        cnt = jnp.sum((ai >= cand).astype(jnp.int32), axis=1, keepdims=True)
        p = jnp.where(cnt >= _TOPK, cand, p)

    gt = ai > p
    cnt_gt = jnp.sum(gt.astype(jnp.int32), axis=1, keepdims=True)
    need = _TOPK - cnt_gt
    eq = ai == p
    idx = lax.broadcasted_iota(jnp.int32, (b, d_lat), 1)

    nbits = max(1, (d_lat - 1).bit_length())
    t = jnp.zeros((b, 1), jnp.int32)
    for bit in range(nbits - 1, -1, -1):
        test = t | jnp.int32(1 << bit)
        cnt = jnp.sum((eq & (idx < test)).astype(jnp.int32), axis=1, keepdims=True)
        t = jnp.where(cnt < need, test, t)

    mask = gt | (eq & (idx <= t))
    zs_ref[...] = jnp.where(mask, z, 0.0)


def _decode_kernel(zs_ref, wd_ref, bd_ref, recon_ref):
    recon_ref[...] = (
        jnp.dot(zs_ref[...], wd_ref[...], preferred_element_type=jnp.float32)
        + bd_ref[...]
    )


def kernel(x, W_enc, b_enc, W_dec, b_dec):
    n_tok, d_in = x.shape
    d_lat = W_enc.shape[1]
    be2 = b_enc.reshape(1, d_lat)
    bd2 = b_dec.reshape(1, d_in)

    b1 = min(_ENC_ROWS, n_tok)
    zs = pl.pallas_call(
        _encode_topk_kernel,
        grid=(n_tok // b1,),
        in_specs=[
            pl.BlockSpec((b1, d_in), lambda i: (i, 0)),
            pl.BlockSpec((d_in, d_lat), lambda i: (0, 0)),
            pl.BlockSpec((1, d_lat), lambda i: (0, 0)),
        ],
        out_specs=pl.BlockSpec((b1, d_lat), lambda i: (i, 0)),
        out_shape=jax.ShapeDtypeStruct((n_tok, d_lat), jnp.float32),
        compiler_params=pltpu.CompilerParams(
            dimension_semantics=("arbitrary",),
        ),
    )(x, W_enc, be2)

    b2 = min(_DEC_ROWS, n_tok)
    recon = pl.pallas_call(
        _decode_kernel,
        grid=(n_tok // b2,),
        in_specs=[
            pl.BlockSpec((b2, d_lat), lambda i: (i, 0)),
            pl.BlockSpec((d_lat, d_in), lambda i: (0, 0)),
            pl.BlockSpec((1, d_in), lambda i: (0, 0)),
        ],
        out_specs=pl.BlockSpec((b2, d_in), lambda i: (i, 0)),
        out_shape=jax.ShapeDtypeStruct((n_tok, d_in), jnp.float32),
        compiler_params=pltpu.CompilerParams(
            dimension_semantics=("arbitrary",),
        ),
    )(zs, W_dec, bd2)
    return (recon, zs)


# speedup vs baseline: 7.5396x; 1.4274x over previous
import jax
import jax.numpy as jnp
from jax import lax
from jax.experimental import pallas as pl
from jax.experimental.pallas import tpu as pltpu

_TOPK = 64
_ENC_ROWS = 256
_ENC_CHUNK = 1024
_DEC_ROWS = 64


def _encode_topk_kernel(x_ref, we_ref, be_ref, zs_ref, ai_ref):
    j = pl.program_id(1)
    n_chunks = pl.num_programs(1)
    b = x_ref.shape[0]
    lb = we_ref.shape[1]
    d_lat = zs_ref.shape[1]

    z = jnp.dot(x_ref[...], we_ref[...], preferred_element_type=jnp.float32)
    z = z + be_ref[...]
    zs_ref[:, pl.ds(j * lb, lb)] = z
    ai_ref[:, pl.ds(j * lb, lb)] = lax.bitcast_convert_type(jnp.abs(z), jnp.int32)

    @pl.when(j == n_chunks - 1)
    def _select():
        def count_ge(cand):
            def body(c, acc):
                a = ai_ref[:, pl.ds(c * lb, lb)]
                return acc + jnp.sum(
                    (a >= cand).astype(jnp.int32), axis=1, keepdims=True
                )

            return lax.fori_loop(0, n_chunks, body, jnp.zeros((b, 1), jnp.int32))

        def val_bit(i, p):
            cand = p | (jnp.int32(1) << (30 - i))
            return jnp.where(count_ge(cand) >= _TOPK, cand, p)

        p = lax.fori_loop(0, 31, val_bit, jnp.zeros((b, 1), jnp.int32))

        cnt_ge = count_ge(p)
        cnt_gt = count_ge(p + 1)
        need = _TOPK - cnt_gt

        def idx_search(_):
            nbits = max(1, (d_lat - 1).bit_length())

            def idx_bit(i, t):
                test = t | (jnp.int32(1) << (nbits - 1 - i))

                def body(c, acc):
                    a = ai_ref[:, pl.ds(c * lb, lb)]
                    base = c * lb
                    idx = base + lax.broadcasted_iota(jnp.int32, (b, lb), 1)
                    hit = (a == p) & (idx < test)
                    return acc + jnp.sum(hit.astype(jnp.int32), axis=1, keepdims=True)

                cnt = lax.fori_loop(0, n_chunks, body, jnp.zeros((b, 1), jnp.int32))
                return jnp.where(cnt < need, test, t)

            return lax.fori_loop(0, nbits, idx_bit, jnp.zeros((b, 1), jnp.int32))

        t = lax.cond(
            jnp.any(cnt_ge > _TOPK),
            idx_search,
            lambda _: jnp.full((b, 1), d_lat, jnp.int32),
            operand=None,
        )

        def mask_chunk(c, _):
            a = ai_ref[:, pl.ds(c * lb, lb)]
            base = c * lb
            idx = base + lax.broadcasted_iota(jnp.int32, (b, lb), 1)
            keep = (a > p) | ((a == p) & (idx <= t))
            zc = zs_ref[:, pl.ds(c * lb, lb)]
            zs_ref[:, pl.ds(c * lb, lb)] = jnp.where(keep, zc, 0.0)
            return 0

        lax.fori_loop(0, n_chunks, mask_chunk, 0)


def _decode_kernel(zs_ref, wd_ref, bd_ref, recon_ref):
    recon_ref[...] = (
        jnp.dot(zs_ref[...], wd_ref[...], preferred_element_type=jnp.float32)
        + bd_ref[...]
    )


def kernel(x, W_enc, b_enc, W_dec, b_dec):
    n_tok, d_in = x.shape
    d_lat = W_enc.shape[1]
    be2 = b_enc.reshape(1, d_lat)
    bd2 = b_dec.reshape(1, d_in)

    b1 = min(_ENC_ROWS, n_tok)
    lb = min(_ENC_CHUNK, d_lat)
    zs = pl.pallas_call(
        _encode_topk_kernel,
        grid=(n_tok // b1, d_lat // lb),
        in_specs=[
            pl.BlockSpec((b1, d_in), lambda i, j: (i, 0)),
            pl.BlockSpec((d_in, lb), lambda i, j: (0, j)),
            pl.BlockSpec((1, lb), lambda i, j: (0, j)),
        ],
        out_specs=pl.BlockSpec((b1, d_lat), lambda i, j: (i, 0)),
        out_shape=jax.ShapeDtypeStruct((n_tok, d_lat), jnp.float32),
        scratch_shapes=[pltpu.VMEM((b1, d_lat), jnp.int32)],
        compiler_params=pltpu.CompilerParams(
            dimension_semantics=("arbitrary", "arbitrary"),
        ),
    )(x, W_enc, be2)

    b2 = min(_DEC_ROWS, n_tok)
    recon = pl.pallas_call(
        _decode_kernel,
        grid=(n_tok // b2,),
        in_specs=[
            pl.BlockSpec((b2, d_lat), lambda i: (i, 0)),
            pl.BlockSpec((d_lat, d_in), lambda i: (0, 0)),
            pl.BlockSpec((1, d_in), lambda i: (0, 0)),
        ],
        out_specs=pl.BlockSpec((b2, d_in), lambda i: (i, 0)),
        out_shape=jax.ShapeDtypeStruct((n_tok, d_in), jnp.float32),
        compiler_params=pltpu.CompilerParams(
            dimension_semantics=("arbitrary",),
        ),
    )(zs, W_dec, bd2)
    return (recon, zs)
